# baseline (speedup 1.0000x reference)
import functools

import jax
import jax.numpy as jnp
from jax import lax
from jax.experimental import pallas as pl
from jax.experimental.pallas import tpu as pltpu
from jax.experimental.pallas import tpu_sc as plsc

_N = 50000
_E = 800000
_G = 128
_NLAYERS = 10
_KCHEB = 10
_FP = 64
_BN = 2000
_NBLK = _N // _BN

_NC = 2
_NS = 16
_CH = 80
_ND = 4
_EPT = _E // _NS
_NSUP = _EPT // _CH
_HALFN = _N // _NC
_ACC_ROWS = 25008
_ZB = 78
_WPT = 1560


def _spmm_body(src_h, dst_h, m_h, out_h, *sc):
    sidx = sc[0:_ND]
    didx = sc[_ND:2 * _ND]
    ldst = sc[2 * _ND:3 * _ND]
    gbuf = sc[3 * _ND:4 * _ND]
    zbuf, acc, isem, gsem, ssem = sc[4 * _ND:]
    c = lax.axis_index("c")
    s = lax.axis_index("s")
    coff = c * _HALFN

    zero16 = jnp.zeros((16,), jnp.float32)

    def _zrow(i, carry):
        for j in range(4):
            zbuf[i, pl.ds(j * 16, 16)] = zero16
        return carry

    lax.fori_loop(0, _ZB, _zrow, 0)
    z0 = s * _WPT
    for k in range(_WPT // _ZB):
        pltpu.sync_copy(zbuf, acc.at[pl.ds(z0 + k * _ZB, _ZB)])

    @pl.when(s == 0)
    def _():
        pltpu.sync_copy(zbuf.at[pl.ds(0, _ACC_ROWS - 16 * _WPT)],
                        acc.at[pl.ds(16 * _WPT, _ACC_ROWS - 16 * _WPT)])

    plsc.subcore_barrier()

    ebase = s * _EPT

    def fire_idx(i, r):
        e0 = pl.multiple_of(ebase + i * _CH, 8)
        pltpu.async_copy(src_h.at[pl.ds(e0, _CH)], sidx[r], isem)
        pltpu.async_copy(dst_h.at[pl.ds(e0, _CH)], didx[r], isem)

    def wait_idx(i, r):
        e0 = pl.multiple_of(ebase + i * _CH, 8)
        pltpu.make_async_copy(src_h.at[pl.ds(e0, _CH)], sidx[r], isem).wait()
        pltpu.make_async_copy(dst_h.at[pl.ds(e0, _CH)], didx[r], isem).wait()

    def compute_ldst(r):
        def _cb(k, carry):
            d = didx[r][pl.ds(k * 16, 16)]
            u = d - coff
            ok = (u >= 0) & (u < _HALFN)
            ldst[r][pl.ds(k * 16, 16)] = jnp.where(ok, u, _HALFN + (d & 7))
            return carry
        lax.fori_loop(0, _CH // 16, _cb, 0)

    def fire_gather(r):
        pltpu.async_copy(m_h.at[sidx[r]], gbuf[r], gsem)

    def wait_gather(r):
        pltpu.make_async_copy(m_h.at[sidx[r]], gbuf[r], gsem).wait()

    def fire_scatter(r):
        pltpu.async_copy(gbuf[r], acc.at[ldst[r]], ssem, add=True)

    def wait_scatter(r):
        pltpu.make_async_copy(gbuf[r], acc.at[ldst[r]], ssem).wait()

    def step(i, p, guard):
        def _s1():
            wait_scatter((p + 1) % _ND)
        def _s2():
            fire_idx(i + 2, (p + 2) % _ND)
        def _s5():
            wait_idx(i + 1, (p + 1) % _ND)
            compute_ldst((p + 1) % _ND)
            fire_gather((p + 1) % _ND)
        if guard:
            pl.when(i >= 3)(_s1)
            pl.when(i + 2 < _NSUP)(_s2)
        else:
            _s1()
            _s2()
        wait_gather(p)
        fire_scatter(p)
        if guard:
            pl.when(i + 1 < _NSUP)(_s5)
        else:
            _s5()

    fire_idx(0, 0)
    fire_idx(1, 1)
    wait_idx(0, 0)
    compute_ldst(0)
    fire_gather(0)

    def quad(q, carry):
        i0 = 4 * q
        for p in range(4):
            step(i0 + p, p, True)
        return carry

    lax.fori_loop(0, _NSUP // 4, quad, 0)
    step(_NSUP - 1, (_NSUP - 1) % _ND, True)
    wait_scatter((_NSUP - 3) % _ND)
    wait_scatter((_NSUP - 2) % _ND)
    wait_scatter((_NSUP - 1) % _ND)

    plsc.subcore_barrier()
    w0 = s * _WPT
    pltpu.sync_copy(acc.at[pl.ds(w0, _WPT)],
                    out_h.at[pl.ds(coff + w0, _WPT)])

    @pl.when(s == 0)
    def _():
        pltpu.sync_copy(acc.at[pl.ds(16 * _WPT, _HALFN - 16 * _WPT)],
                        out_h.at[pl.ds(coff + 16 * _WPT, _HALFN - 16 * _WPT)])


_sc_mesh = plsc.VectorSubcoreMesh(core_axis_name="c", subcore_axis_name="s",
                                  num_cores=_NC, num_subcores=_NS)

_spmm_call = functools.partial(
    pl.kernel,
    out_type=jax.ShapeDtypeStruct((_N, _FP), jnp.float32),
    mesh=_sc_mesh,
    compiler_params=pltpu.CompilerParams(use_tc_tiling_on_sc=False),
    scratch_types=(
        [pltpu.VMEM((_CH,), jnp.int32)] * (3 * _ND)
        + [pltpu.VMEM((_CH, _FP), jnp.float32)] * _ND
        + [pltpu.VMEM((_ZB, _FP), jnp.float32),
           pltpu.VMEM_SHARED((_ACC_ROWS, _FP), jnp.float32),
           pltpu.SemaphoreType.DMA,
           pltpu.SemaphoreType.DMA,
           pltpu.SemaphoreType.DMA]
    ),
)(_spmm_body)


def _spmm(src1, dst1, m):
    return _spmm_call(src1, dst1, m)


def _vspec(w=_FP):
    return pl.BlockSpec((_BN, w), lambda i: (i, 0))


def _wspec(shape):
    return pl.BlockSpec(shape, lambda i: (0, 0))


def _k1_body(x_ref, deg_ref, w1, b1, w2, b2, w3, b3,
             h1_ref, h2_ref, h3_ref, dis_ref, dinv_ref):
    xb = x_ref[...]

    def proj(w, b):
        t = jnp.dot(xb, w[...], preferred_element_type=jnp.float32) + b[...]
        return jnp.where(t >= 0, t, 0.01 * t)

    h1_ref[...] = proj(w1, b1)
    h2_ref[...] = proj(w2, b2)
    h3_ref[...] = proj(w3, b3)
    deg0 = deg_ref[...][:, 0:1]
    dis = lax.rsqrt(deg0 + 1.0)
    dinv = jnp.where(deg0 > 0, lax.rsqrt(jnp.maximum(deg0, 1.0)), 0.0)
    dis_ref[...] = jnp.broadcast_to(dis, (_BN, 8))
    dinv_ref[...] = jnp.broadcast_to(dinv, (_BN, 8))


_k1 = pl.pallas_call(
    _k1_body,
    grid=(_NBLK,),
    in_specs=[_vspec(8), _vspec(_FP), _wspec((8, _FP)), _wspec((1, _FP)),
              _wspec((8, _FP)), _wspec((1, _FP)), _wspec((8, _FP)),
              _wspec((1, _FP))],
    out_specs=[_vspec(), _vspec(), _vspec(), _vspec(8), _vspec(8)],
    out_shape=[jax.ShapeDtypeStruct((_N, _FP), jnp.float32)] * 3
    + [jax.ShapeDtypeStruct((_N, 8), jnp.float32)] * 2,
)


def _gcn_a_body(h_ref, w_ref, dis_ref, m_ref, v_ref):
    m = jnp.dot(h_ref[...], w_ref[...], preferred_element_type=jnp.float32)
    m_ref[...] = m
    v_ref[...] = dis_ref[...][:, 0:1] * m


_gcn_a = pl.pallas_call(
    _gcn_a_body,
    grid=(_NBLK,),
    in_specs=[_vspec(), _wspec((_FP, _FP)), _vspec(8)],
    out_specs=[_vspec(), _vspec()],
    out_shape=[jax.ShapeDtypeStruct((_N, _FP), jnp.float32)] * 2,
)


def _gcn_b_body(s_ref, m_ref, dis_ref, b_ref, h_ref):
    d0 = dis_ref[...][:, 0:1]
    h_ref[...] = d0 * s_ref[...] + (d0 * d0) * m_ref[...] + b_ref[...]


_gcn_b = pl.pallas_call(
    _gcn_b_body,
    grid=(_NBLK,),
    in_specs=[_vspec(), _vspec(), _vspec(8), _wspec((1, _FP))],
    out_specs=_vspec(),
    out_shape=jax.ShapeDtypeStruct((_N, _FP), jnp.float32),
)


def _gin_body(s_ref, h_ref, w_ref, b_ref, o_ref):
    t = jnp.dot(h_ref[...] + s_ref[...], w_ref[...],
                preferred_element_type=jnp.float32) + b_ref[...]
    o_ref[...] = t - jnp.tanh(t)


_gin_c = pl.pallas_call(
    _gin_body,
    grid=(_NBLK,),
    in_specs=[_vspec(), _vspec(), _wspec((_FP, _FP)), _wspec((1, _FP))],
    out_specs=_vspec(),
    out_shape=jax.ShapeDtypeStruct((_N, _FP), jnp.float32),
)


def _cheb0_body(t0_ref, w_ref, dinv_ref, o3_ref, v_ref):
    t0 = t0_ref[...]
    o3_ref[...] = jnp.dot(t0, w_ref[...], preferred_element_type=jnp.float32)
    v_ref[...] = dinv_ref[...][:, 0:1] * t0


_cheb0 = pl.pallas_call(
    _cheb0_body,
    grid=(_NBLK,),
    in_specs=[_vspec(), _wspec((_FP, _FP)), _vspec(8)],
    out_specs=[_vspec(), _vspec()],
    out_shape=[jax.ShapeDtypeStruct((_N, _FP), jnp.float32)] * 2,
)


def _cheb1_body(s_ref, o3in_ref, w_ref, dinv_ref, o3_ref, t1_ref, v_ref):
    d0 = dinv_ref[...][:, 0:1]
    t1 = -d0 * s_ref[...]
    o3_ref[...] = o3in_ref[...] + jnp.dot(t1, w_ref[...],
                                          preferred_element_type=jnp.float32)
    t1_ref[...] = t1
    v_ref[...] = d0 * t1


_cheb1 = pl.pallas_call(
    _cheb1_body,
    grid=(_NBLK,),
    in_specs=[_vspec(), _vspec(), _wspec((_FP, _FP)), _vspec(8)],
    out_specs=[_vspec(), _vspec(), _vspec()],
    out_shape=[jax.ShapeDtypeStruct((_N, _FP), jnp.float32)] * 3,
)


def _chebk_body(s_ref, tp_ref, o3in_ref, w_ref, dinv_ref,
                o3_ref, tk_ref, v_ref):
    d0 = dinv_ref[...][:, 0:1]
    tk = -2.0 * d0 * s_ref[...] - tp_ref[...]
    o3_ref[...] = o3in_ref[...] + jnp.dot(tk, w_ref[...],
                                          preferred_element_type=jnp.float32)
    tk_ref[...] = tk
    v_ref[...] = d0 * tk


_chebk = pl.pallas_call(
    _chebk_body,
    grid=(_NBLK,),
    in_specs=[_vspec(), _vspec(), _vspec(), _wspec((_FP, _FP)), _vspec(8)],
    out_specs=[_vspec(), _vspec(), _vspec()],
    out_shape=[jax.ShapeDtypeStruct((_N, _FP), jnp.float32)] * 3,
)


def _cheb9_body(s_ref, tp_ref, o3in_ref, w_ref, dinv_ref, cb_ref, o3_ref):
    d0 = dinv_ref[...][:, 0:1]
    t9 = -2.0 * d0 * s_ref[...] - tp_ref[...]
    o3_ref[...] = (o3in_ref[...]
                   + jnp.dot(t9, w_ref[...], preferred_element_type=jnp.float32)
                   + cb_ref[...])


_cheb9 = pl.pallas_call(
    _cheb9_body,
    grid=(_NBLK,),
    in_specs=[_vspec(), _vspec(), _vspec(), _wspec((_FP, _FP)), _vspec(8),
              _wspec((1, _FP))],
    out_specs=_vspec(),
    out_shape=jax.ShapeDtypeStruct((_N, _FP), jnp.float32),
)


def _pool_body(h_ref, b_ref, add_ref, cnt_ref, mx_ref):
    gb = pl.program_id(0)
    nb = pl.program_id(1)

    @pl.when(nb == 0)
    def _():
        add_ref[...] = jnp.zeros_like(add_ref)
        cnt_ref[...] = jnp.zeros_like(cnt_ref)
        mx_ref[...] = jnp.full_like(mx_ref, -1e30)

    h = h_ref[...]
    bb = b_ref[...]
    gids = gb * 8 + lax.broadcasted_iota(jnp.int32, (1, 8), 1)
    p = bb == gids
    pf = p.astype(jnp.float32)
    add_ref[...] += lax.dot_general(pf, h, (((0,), (0,)), ((), ())),
                                    preferred_element_type=jnp.float32)
    cnt_ref[...] += lax.dot_general(pf, jnp.ones((_BN, 1), jnp.float32),
                                    (((0,), (0,)), ((), ())),
                                    preferred_element_type=jnp.float32)
    for j in range(8):
        mj = p[:, j:j + 1]
        cand = jnp.max(jnp.where(mj, h, -1e30), axis=0, keepdims=True)
        mx_ref[j:j + 1, :] = jnp.maximum(mx_ref[j:j + 1, :], cand)


_pool = pl.pallas_call(
    _pool_body,
    grid=(_G // 8, _NBLK),
    in_specs=[pl.BlockSpec((_BN, _FP), lambda g, i: (i, 0)),
              pl.BlockSpec((_BN, 1), lambda g, i: (i, 0))],
    out_specs=[pl.BlockSpec((8, _FP), lambda g, i: (g, 0)),
               pl.BlockSpec((8, 1), lambda g, i: (g, 0)),
               pl.BlockSpec((8, _FP), lambda g, i: (g, 0))],
    out_shape=[jax.ShapeDtypeStruct((_G, _FP), jnp.float32),
               jax.ShapeDtypeStruct((_G, 1), jnp.float32),
               jax.ShapeDtypeStruct((_G, _FP), jnp.float32)],
)


def _fin_body(a1, c1, m1, a2, c2, m2, a3, c3, m3, fw, fb, out_ref):
    def part(a_ref, c_ref, m_ref):
        a = a_ref[...]
        c = c_ref[...]
        mean = a / jnp.maximum(c, 1.0)
        mx = jnp.where(c > 0, m_ref[...], 0.0)
        return jnp.concatenate([a, mean, mx], axis=1)

    feats = jnp.concatenate([part(a1, c1, m1), part(a2, c2, m2),
                             part(a3, c3, m3)], axis=1)
    logits = jnp.dot(feats, fw[...], preferred_element_type=jnp.float32) \
        + fb[...]
    mxl = jnp.max(logits, axis=-1, keepdims=True)
    sh = logits - mxl
    out_ref[...] = sh - jnp.log(jnp.sum(jnp.exp(sh), axis=-1, keepdims=True))


_fin = pl.pallas_call(
    _fin_body,
    out_shape=jax.ShapeDtypeStruct((_G, 1), jnp.float32),
)


def _pad_w(w):
    fi, fo = w.shape
    return jnp.pad(w, ((0, 0), (0, _FP - fo))) if fi == _FP else \
        jnp.pad(w, ((0, 8 - fi), (0, _FP - fo)))


def _pad_b(b):
    return jnp.pad(b, (0, _FP - b.shape[0])).reshape(1, _FP)


def kernel(x, edge_index, batch, lin1_W, lin1_b, gcn_W, gcn_b, lin2_W, lin2_b,
           gin_W, gin_b, lin3_W, lin3_b, cheb_W, cheb_b, fin_W, fin_b):
    src1 = edge_index[0].astype(jnp.int32)
    dst1 = edge_index[1].astype(jnp.int32)
    batch2 = batch.astype(jnp.int32).reshape(_N, 1)
    x_pad = jnp.pad(x, ((0, 0), (0, 8 - x.shape[1])))

    gw = [jnp.pad(gcn_W[i], ((0, 4), (0, 4))) for i in range(_NLAYERS)]
    gb = [_pad_b(gcn_b[i]) for i in range(_NLAYERS)]
    iw = [jnp.pad(gin_W[i], ((0, 4), (0, 4))) for i in range(_NLAYERS)]
    ib = [_pad_b(gin_b[i]) for i in range(_NLAYERS)]
    cw = [jnp.pad(cheb_W[k], ((0, 4), (0, 4))) for k in range(_KCHEB)]
    cbp = _pad_b(cheb_b)
    fwp = jnp.pad(fin_W.reshape(9, 60, 1), ((0, 0), (0, 4), (0, 0))) \
        .reshape(9 * _FP, 1)
    fbp = fin_b.reshape(1, 1)

    ones64 = jnp.ones((_N, _FP), jnp.float32)
    deg64 = _spmm(src1, dst1, ones64)
    h1, h2, h3, dis8, dinv8 = _k1(x_pad, deg64, _pad_w(lin1_W), _pad_b(lin1_b),
                                  _pad_w(lin2_W), _pad_b(lin2_b),
                                  _pad_w(lin3_W), _pad_b(lin3_b))

    h = h1
    for i in range(_NLAYERS):
        m, v = _gcn_a(h, gw[i], dis8)
        agg = _spmm(src1, dst1, v)
        h = _gcn_b(agg, m, dis8, gb[i])
    p1 = _pool(h, batch2)

    h = h2
    for i in range(_NLAYERS):
        agg = _spmm(src1, dst1, h)
        h = _gin_c(agg, h, iw[i], ib[i])
    p2 = _pool(h, batch2)

    o3, v = _cheb0(h3, cw[0], dinv8)
    agg = _spmm(src1, dst1, v)
    o3, t_im1, v = _cheb1(agg, o3, cw[1], dinv8)
    t_im2 = h3
    for k in range(2, _KCHEB):
        agg = _spmm(src1, dst1, v)
        if k < _KCHEB - 1:
            o3, tk, v = _chebk(agg, t_im2, o3, cw[k], dinv8)
            t_im2, t_im1 = t_im1, tk
        else:
            o3 = _cheb9(agg, t_im2, o3, cw[k], dinv8, cbp)
    p3 = _pool(o3, batch2)

    return _fin(p1[0], p1[1], p1[2], p2[0], p2[1], p2[2],
                p3[0], p3[1], p3[2], fwp, fbp)


# speedup vs baseline: 1.0037x; 1.0037x over previous
import functools

import jax
import jax.numpy as jnp
from jax import lax
from jax.experimental import pallas as pl
from jax.experimental.pallas import tpu as pltpu
from jax.experimental.pallas import tpu_sc as plsc

_N = 50000
_E = 800000
_G = 128
_NLAYERS = 10
_KCHEB = 10
_FP = 64
_BN = 2000
_NBLK = _N // _BN

_NC = 2
_NS = 16
_CH = 80
_ND = 4
_EPT = _E // _NS
_NSUP = _EPT // _CH
_HALFN = _N // _NC
_ACC_ROWS = 25008
_ZB = 78
_WPT = 1560


def _spmm_body(eidx_h, m_h, out_h, *sc):
    sdix = sc[0:_ND]
    ldst = sc[_ND:2 * _ND]
    gbuf = sc[2 * _ND:3 * _ND]
    zbuf, acc, isem, gsem, ssem = sc[3 * _ND:]
    c = lax.axis_index("c")
    s = lax.axis_index("s")
    coff = c * _HALFN

    zero16 = jnp.zeros((16,), jnp.float32)

    def _zrow(i, carry):
        for j in range(4):
            zbuf[i, pl.ds(j * 16, 16)] = zero16
        return carry

    lax.fori_loop(0, _ZB, _zrow, 0)
    z0 = s * _WPT
    for k in range(_WPT // _ZB):
        pltpu.sync_copy(zbuf, acc.at[pl.ds(z0 + k * _ZB, _ZB)])

    @pl.when(s == 0)
    def _():
        pltpu.sync_copy(zbuf.at[pl.ds(0, _ACC_ROWS - 16 * _WPT)],
                        acc.at[pl.ds(16 * _WPT, _ACC_ROWS - 16 * _WPT)])

    plsc.subcore_barrier()

    ebase = s * _EPT

    def fire_idx(i, r):
        e0 = pl.multiple_of(ebase + i * _CH, 8)
        pltpu.async_copy(eidx_h.at[:, pl.ds(e0, _CH)], sdix[r], isem)

    def wait_idx(i, r):
        e0 = pl.multiple_of(ebase + i * _CH, 8)
        pltpu.make_async_copy(eidx_h.at[:, pl.ds(e0, _CH)], sdix[r],
                              isem).wait()

    def compute_ldst(r):
        def _cb(k, carry):
            d = sdix[r][1, pl.ds(k * 16, 16)]
            u = d - coff
            ok = (u >= 0) & (u < _HALFN)
            ldst[r][pl.ds(k * 16, 16)] = jnp.where(ok, u, _HALFN + (d & 7))
            return carry
        lax.fori_loop(0, _CH // 16, _cb, 0)

    def fire_gather(r):
        pltpu.async_copy(m_h.at[sdix[r].at[0]], gbuf[r], gsem)

    def wait_gather(r):
        pltpu.make_async_copy(m_h.at[sdix[r].at[0]], gbuf[r], gsem).wait()

    def fire_scatter(r):
        pltpu.async_copy(gbuf[r], acc.at[ldst[r]], ssem, add=True)

    def wait_scatter(r):
        pltpu.make_async_copy(gbuf[r], acc.at[ldst[r]], ssem).wait()

    def step(i, p, guard):
        def _s1():
            wait_scatter((p + 1) % _ND)
        def _s2():
            fire_idx(i + 2, (p + 2) % _ND)
        def _s5():
            wait_idx(i + 1, (p + 1) % _ND)
            compute_ldst((p + 1) % _ND)
            fire_gather((p + 1) % _ND)
        if guard:
            pl.when(i >= 3)(_s1)
            pl.when(i + 2 < _NSUP)(_s2)
        else:
            _s1()
            _s2()
        wait_gather(p)
        fire_scatter(p)
        if guard:
            pl.when(i + 1 < _NSUP)(_s5)
        else:
            _s5()

    fire_idx(0, 0)
    fire_idx(1, 1)
    wait_idx(0, 0)
    compute_ldst(0)
    fire_gather(0)

    def quad(q, carry):
        i0 = 4 * q
        for p in range(4):
            step(i0 + p, p, True)
        return carry

    lax.fori_loop(0, _NSUP // 4, quad, 0)
    step(_NSUP - 1, (_NSUP - 1) % _ND, True)
    wait_scatter((_NSUP - 3) % _ND)
    wait_scatter((_NSUP - 2) % _ND)
    wait_scatter((_NSUP - 1) % _ND)

    plsc.subcore_barrier()
    w0 = s * _WPT
    pltpu.sync_copy(acc.at[pl.ds(w0, _WPT)],
                    out_h.at[pl.ds(coff + w0, _WPT)])

    @pl.when(s == 0)
    def _():
        pltpu.sync_copy(acc.at[pl.ds(16 * _WPT, _HALFN - 16 * _WPT)],
                        out_h.at[pl.ds(coff + 16 * _WPT, _HALFN - 16 * _WPT)])


_sc_mesh = plsc.VectorSubcoreMesh(core_axis_name="c", subcore_axis_name="s",
                                  num_cores=_NC, num_subcores=_NS)

_spmm_call = functools.partial(
    pl.kernel,
    out_type=jax.ShapeDtypeStruct((_N, _FP), jnp.float32),
    mesh=_sc_mesh,
    compiler_params=pltpu.CompilerParams(use_tc_tiling_on_sc=False),
    scratch_types=(
        [pltpu.VMEM((2, _CH), jnp.int32)] * _ND
        + [pltpu.VMEM((_CH,), jnp.int32)] * _ND
        + [pltpu.VMEM((_CH, _FP), jnp.float32)] * _ND
        + [pltpu.VMEM((_ZB, _FP), jnp.float32),
           pltpu.VMEM_SHARED((_ACC_ROWS, _FP), jnp.float32),
           pltpu.SemaphoreType.DMA,
           pltpu.SemaphoreType.DMA,
           pltpu.SemaphoreType.DMA]
    ),
)(_spmm_body)


def _spmm(eidx, m):
    return _spmm_call(eidx, m)


def _vspec(w=_FP):
    return pl.BlockSpec((_BN, w), lambda i: (i, 0))


def _wspec(shape):
    return pl.BlockSpec(shape, lambda i: (0, 0))


def _k1_body(x_ref, deg_ref, w1, b1, w2, b2, w3, b3,
             h1_ref, h2_ref, h3_ref, dis_ref, dinv_ref):
    xb = x_ref[...]

    def proj(w, b):
        t = jnp.dot(xb, w[...], preferred_element_type=jnp.float32) + b[...]
        return jnp.where(t >= 0, t, 0.01 * t)

    h1_ref[...] = proj(w1, b1)
    h2_ref[...] = proj(w2, b2)
    h3_ref[...] = proj(w3, b3)
    deg0 = deg_ref[...][:, 0:1]
    dis = lax.rsqrt(deg0 + 1.0)
    dinv = jnp.where(deg0 > 0, lax.rsqrt(jnp.maximum(deg0, 1.0)), 0.0)
    dis_ref[...] = jnp.broadcast_to(dis, (_BN, 8))
    dinv_ref[...] = jnp.broadcast_to(dinv, (_BN, 8))


_k1 = pl.pallas_call(
    _k1_body,
    grid=(_NBLK,),
    in_specs=[_vspec(8), _vspec(_FP), _wspec((8, _FP)), _wspec((1, _FP)),
              _wspec((8, _FP)), _wspec((1, _FP)), _wspec((8, _FP)),
              _wspec((1, _FP))],
    out_specs=[_vspec(), _vspec(), _vspec(), _vspec(8), _vspec(8)],
    out_shape=[jax.ShapeDtypeStruct((_N, _FP), jnp.float32)] * 3
    + [jax.ShapeDtypeStruct((_N, 8), jnp.float32)] * 2,
)


def _gcn_a_body(h_ref, w_ref, dis_ref, m_ref, v_ref):
    m = jnp.dot(h_ref[...], w_ref[...], preferred_element_type=jnp.float32)
    m_ref[...] = m
    v_ref[...] = dis_ref[...][:, 0:1] * m


_gcn_a = pl.pallas_call(
    _gcn_a_body,
    grid=(_NBLK,),
    in_specs=[_vspec(), _wspec((_FP, _FP)), _vspec(8)],
    out_specs=[_vspec(), _vspec()],
    out_shape=[jax.ShapeDtypeStruct((_N, _FP), jnp.float32)] * 2,
)


def _gcn_b_body(s_ref, m_ref, dis_ref, b_ref, h_ref):
    d0 = dis_ref[...][:, 0:1]
    h_ref[...] = d0 * s_ref[...] + (d0 * d0) * m_ref[...] + b_ref[...]


_gcn_b = pl.pallas_call(
    _gcn_b_body,
    grid=(_NBLK,),
    in_specs=[_vspec(), _vspec(), _vspec(8), _wspec((1, _FP))],
    out_specs=_vspec(),
    out_shape=jax.ShapeDtypeStruct((_N, _FP), jnp.float32),
)


def _gin_body(s_ref, h_ref, w_ref, b_ref, o_ref):
    t = jnp.dot(h_ref[...] + s_ref[...], w_ref[...],
                preferred_element_type=jnp.float32) + b_ref[...]
    o_ref[...] = t - jnp.tanh(t)


_gin_c = pl.pallas_call(
    _gin_body,
    grid=(_NBLK,),
    in_specs=[_vspec(), _vspec(), _wspec((_FP, _FP)), _wspec((1, _FP))],
    out_specs=_vspec(),
    out_shape=jax.ShapeDtypeStruct((_N, _FP), jnp.float32),
)


def _cheb0_body(t0_ref, w_ref, dinv_ref, o3_ref, v_ref):
    t0 = t0_ref[...]
    o3_ref[...] = jnp.dot(t0, w_ref[...], preferred_element_type=jnp.float32)
    v_ref[...] = dinv_ref[...][:, 0:1] * t0


_cheb0 = pl.pallas_call(
    _cheb0_body,
    grid=(_NBLK,),
    in_specs=[_vspec(), _wspec((_FP, _FP)), _vspec(8)],
    out_specs=[_vspec(), _vspec()],
    out_shape=[jax.ShapeDtypeStruct((_N, _FP), jnp.float32)] * 2,
)


def _cheb1_body(s_ref, o3in_ref, w_ref, dinv_ref, o3_ref, t1_ref, v_ref):
    d0 = dinv_ref[...][:, 0:1]
    t1 = -d0 * s_ref[...]
    o3_ref[...] = o3in_ref[...] + jnp.dot(t1, w_ref[...],
                                          preferred_element_type=jnp.float32)
    t1_ref[...] = t1
    v_ref[...] = d0 * t1


_cheb1 = pl.pallas_call(
    _cheb1_body,
    grid=(_NBLK,),
    in_specs=[_vspec(), _vspec(), _wspec((_FP, _FP)), _vspec(8)],
    out_specs=[_vspec(), _vspec(), _vspec()],
    out_shape=[jax.ShapeDtypeStruct((_N, _FP), jnp.float32)] * 3,
)


def _chebk_body(s_ref, tp_ref, o3in_ref, w_ref, dinv_ref,
                o3_ref, tk_ref, v_ref):
    d0 = dinv_ref[...][:, 0:1]
    tk = -2.0 * d0 * s_ref[...] - tp_ref[...]
    o3_ref[...] = o3in_ref[...] + jnp.dot(tk, w_ref[...],
                                          preferred_element_type=jnp.float32)
    tk_ref[...] = tk
    v_ref[...] = d0 * tk


_chebk = pl.pallas_call(
    _chebk_body,
    grid=(_NBLK,),
    in_specs=[_vspec(), _vspec(), _vspec(), _wspec((_FP, _FP)), _vspec(8)],
    out_specs=[_vspec(), _vspec(), _vspec()],
    out_shape=[jax.ShapeDtypeStruct((_N, _FP), jnp.float32)] * 3,
)


def _cheb9_body(s_ref, tp_ref, o3in_ref, w_ref, dinv_ref, cb_ref, o3_ref):
    d0 = dinv_ref[...][:, 0:1]
    t9 = -2.0 * d0 * s_ref[...] - tp_ref[...]
    o3_ref[...] = (o3in_ref[...]
                   + jnp.dot(t9, w_ref[...], preferred_element_type=jnp.float32)
                   + cb_ref[...])


_cheb9 = pl.pallas_call(
    _cheb9_body,
    grid=(_NBLK,),
    in_specs=[_vspec(), _vspec(), _vspec(), _wspec((_FP, _FP)), _vspec(8),
              _wspec((1, _FP))],
    out_specs=_vspec(),
    out_shape=jax.ShapeDtypeStruct((_N, _FP), jnp.float32),
)


def _pool_body(h_ref, b_ref, add_ref, cnt_ref, mx_ref):
    gb = pl.program_id(0)
    nb = pl.program_id(1)

    @pl.when(nb == 0)
    def _():
        add_ref[...] = jnp.zeros_like(add_ref)
        cnt_ref[...] = jnp.zeros_like(cnt_ref)
        mx_ref[...] = jnp.full_like(mx_ref, -1e30)

    h = h_ref[...]
    bb = b_ref[...]
    gids = gb * 8 + lax.broadcasted_iota(jnp.int32, (1, 8), 1)
    p = bb == gids
    pf = p.astype(jnp.float32)
    add_ref[...] += lax.dot_general(pf, h, (((0,), (0,)), ((), ())),
                                    preferred_element_type=jnp.float32)
    cnt_ref[...] += lax.dot_general(pf, jnp.ones((_BN, 1), jnp.float32),
                                    (((0,), (0,)), ((), ())),
                                    preferred_element_type=jnp.float32)
    for j in range(8):
        mj = p[:, j:j + 1]
        cand = jnp.max(jnp.where(mj, h, -1e30), axis=0, keepdims=True)
        mx_ref[j:j + 1, :] = jnp.maximum(mx_ref[j:j + 1, :], cand)


_pool = pl.pallas_call(
    _pool_body,
    grid=(_G // 8, _NBLK),
    in_specs=[pl.BlockSpec((_BN, _FP), lambda g, i: (i, 0)),
              pl.BlockSpec((_BN, 1), lambda g, i: (i, 0))],
    out_specs=[pl.BlockSpec((8, _FP), lambda g, i: (g, 0)),
               pl.BlockSpec((8, 1), lambda g, i: (g, 0)),
               pl.BlockSpec((8, _FP), lambda g, i: (g, 0))],
    out_shape=[jax.ShapeDtypeStruct((_G, _FP), jnp.float32),
               jax.ShapeDtypeStruct((_G, 1), jnp.float32),
               jax.ShapeDtypeStruct((_G, _FP), jnp.float32)],
)


def _fin_body(a1, c1, m1, a2, c2, m2, a3, c3, m3, fw, fb, out_ref):
    def part(a_ref, c_ref, m_ref):
        a = a_ref[...]
        c = c_ref[...]
        mean = a / jnp.maximum(c, 1.0)
        mx = jnp.where(c > 0, m_ref[...], 0.0)
        return jnp.concatenate([a, mean, mx], axis=1)

    feats = jnp.concatenate([part(a1, c1, m1), part(a2, c2, m2),
                             part(a3, c3, m3)], axis=1)
    logits = jnp.dot(feats, fw[...], preferred_element_type=jnp.float32) \
        + fb[...]
    mxl = jnp.max(logits, axis=-1, keepdims=True)
    sh = logits - mxl
    out_ref[...] = sh - jnp.log(jnp.sum(jnp.exp(sh), axis=-1, keepdims=True))


_fin = pl.pallas_call(
    _fin_body,
    out_shape=jax.ShapeDtypeStruct((_G, 1), jnp.float32),
)


def _pad_w(w):
    fi, fo = w.shape
    return jnp.pad(w, ((0, 0), (0, _FP - fo))) if fi == _FP else \
        jnp.pad(w, ((0, 8 - fi), (0, _FP - fo)))


def _pad_b(b):
    return jnp.pad(b, (0, _FP - b.shape[0])).reshape(1, _FP)


def kernel(x, edge_index, batch, lin1_W, lin1_b, gcn_W, gcn_b, lin2_W, lin2_b,
           gin_W, gin_b, lin3_W, lin3_b, cheb_W, cheb_b, fin_W, fin_b):
    eidx = edge_index.astype(jnp.int32)
    batch2 = batch.astype(jnp.int32).reshape(_N, 1)
    x_pad = jnp.pad(x, ((0, 0), (0, 8 - x.shape[1])))

    gw = [jnp.pad(gcn_W[i], ((0, 4), (0, 4))) for i in range(_NLAYERS)]
    gb = [_pad_b(gcn_b[i]) for i in range(_NLAYERS)]
    iw = [jnp.pad(gin_W[i], ((0, 4), (0, 4))) for i in range(_NLAYERS)]
    ib = [_pad_b(gin_b[i]) for i in range(_NLAYERS)]
    cw = [jnp.pad(cheb_W[k], ((0, 4), (0, 4))) for k in range(_KCHEB)]
    cbp = _pad_b(cheb_b)
    fwp = jnp.pad(fin_W.reshape(9, 60, 1), ((0, 0), (0, 4), (0, 0))) \
        .reshape(9 * _FP, 1)
    fbp = fin_b.reshape(1, 1)

    ones64 = jnp.ones((_N, _FP), jnp.float32)
    deg64 = _spmm(eidx, ones64)
    h1, h2, h3, dis8, dinv8 = _k1(x_pad, deg64, _pad_w(lin1_W), _pad_b(lin1_b),
                                  _pad_w(lin2_W), _pad_b(lin2_b),
                                  _pad_w(lin3_W), _pad_b(lin3_b))

    h = h1
    for i in range(_NLAYERS):
        m, v = _gcn_a(h, gw[i], dis8)
        agg = _spmm(eidx, v)
        h = _gcn_b(agg, m, dis8, gb[i])
    p1 = _pool(h, batch2)

    h = h2
    for i in range(_NLAYERS):
        agg = _spmm(eidx, h)
        h = _gin_c(agg, h, iw[i], ib[i])
    p2 = _pool(h, batch2)

    o3, v = _cheb0(h3, cw[0], dinv8)
    agg = _spmm(eidx, v)
    o3, t_im1, v = _cheb1(agg, o3, cw[1], dinv8)
    t_im2 = h3
    for k in range(2, _KCHEB):
        agg = _spmm(eidx, v)
        if k < _KCHEB - 1:
            o3, tk, v = _chebk(agg, t_im2, o3, cw[k], dinv8)
            t_im2, t_im1 = t_im1, tk
        else:
            o3 = _cheb9(agg, t_im2, o3, cw[k], dinv8, cbp)
    p3 = _pool(o3, batch2)

    return _fin(p1[0], p1[1], p1[2], p2[0], p2[1], p2[2],
                p3[0], p3[1], p3[2], fwp, fbp)


# speedup vs baseline: 1.2416x; 1.2371x over previous
import functools

import jax
import jax.numpy as jnp
from jax import lax
from jax.experimental import pallas as pl
from jax.experimental.pallas import tpu as pltpu
from jax.experimental.pallas import tpu_sc as plsc

_N = 50000
_E = 800000
_G = 128
_NLAYERS = 10
_KCHEB = 10
_FP = 64
_BN = 2000
_NBLK = _N // _BN

_NC = 2
_NS = 16
_CH = 128
_ND = 3
_EPT = _E // _NS
_NSUP = _EPT // _CH
_TAIL = _EPT - _NSUP * _CH
_HALFN = _N // _NC
_ACC_ROWS = 25008
_ZB = 65
_WPT = 1560


def _spmm_body(eidx_h, m_h, out_h, *sc):
    sdix = sc[0:_ND]
    ldst = sc[_ND:2 * _ND]
    gbuf = sc[2 * _ND:3 * _ND]
    tidx, tldst, zbuf, acc, isem, gsem, ssem = sc[3 * _ND:]
    c = lax.axis_index("c")
    s = lax.axis_index("s")
    coff = c * _HALFN

    zero16 = jnp.zeros((16,), jnp.float32)

    def _zrow(i, carry):
        for j in range(4):
            zbuf[i, pl.ds(j * 16, 16)] = zero16
        return carry

    lax.fori_loop(0, _ZB, _zrow, 0)
    z0 = s * _WPT
    for k in range(_WPT // _ZB):
        pltpu.sync_copy(zbuf, acc.at[pl.ds(z0 + k * _ZB, _ZB)])

    @pl.when(s == 0)
    def _():
        pltpu.sync_copy(zbuf.at[pl.ds(0, _ACC_ROWS - 16 * _WPT)],
                        acc.at[pl.ds(16 * _WPT, _ACC_ROWS - 16 * _WPT)])

    plsc.subcore_barrier()

    ebase = s * _EPT

    def fire_idx(i, r):
        e0 = pl.multiple_of(ebase + i * _CH, 8)
        pltpu.async_copy(eidx_h.at[:, pl.ds(e0, _CH)], sdix[r], isem)

    def wait_idx(i, r):
        e0 = pl.multiple_of(ebase + i * _CH, 8)
        pltpu.make_async_copy(eidx_h.at[:, pl.ds(e0, _CH)], sdix[r],
                              isem).wait()

    def compute_ldst(r):
        def _cb(k, carry):
            d = sdix[r][1, pl.ds(k * 16, 16)]
            u = d - coff
            ok = (u >= 0) & (u < _HALFN)
            ldst[r][pl.ds(k * 16, 16)] = jnp.where(ok, u, _HALFN + (d & 7))
            return carry
        lax.fori_loop(0, _CH // 16, _cb, 0)

    def fire_gather(r):
        pltpu.async_copy(m_h.at[sdix[r].at[0]], gbuf[r], gsem)

    def wait_gather(r):
        pltpu.make_async_copy(m_h.at[sdix[r].at[0]], gbuf[r], gsem).wait()

    def fire_scatter(r):
        pltpu.async_copy(gbuf[r], acc.at[ldst[r]], ssem, add=True)

    def wait_scatter(r):
        pltpu.make_async_copy(gbuf[r], acc.at[ldst[r]], ssem).wait()

    def step(i, p):
        pl.when(i >= _ND - 1)(lambda: wait_scatter((p + 1) % _ND))
        pl.when(i + 2 < _NSUP)(lambda: fire_idx(i + 2, (p + 2) % _ND))
        wait_gather(p)
        fire_scatter(p)

        def _s5():
            wait_idx(i + 1, (p + 1) % _ND)
            compute_ldst((p + 1) % _ND)
            fire_gather((p + 1) % _ND)

        pl.when(i + 1 < _NSUP)(_s5)

    fire_idx(0, 0)
    fire_idx(1, 1)
    wait_idx(0, 0)
    compute_ldst(0)
    fire_gather(0)

    def triple(q, carry):
        i0 = 3 * q
        for p in range(3):
            step(i0 + p, p)
        return carry

    lax.fori_loop(0, _NSUP // 3, triple, 0)
    wait_scatter((_NSUP - 2) % _ND)
    wait_scatter((_NSUP - 1) % _ND)

    et = pl.multiple_of(ebase + _NSUP * _CH, 8)
    pltpu.async_copy(eidx_h.at[:, pl.ds(et, _TAIL)], tidx, isem).wait()

    def _tcb(k, carry):
        d = tidx[1, pl.ds(k * 16, 16)]
        u = d - coff
        ok = (u >= 0) & (u < _HALFN)
        tldst[pl.ds(k * 16, 16)] = jnp.where(ok, u, _HALFN + (d & 7))
        return carry

    lax.fori_loop(0, _TAIL // 16, _tcb, 0)
    pltpu.async_copy(m_h.at[tidx.at[0]], gbuf[0].at[pl.ds(0, _TAIL)],
                     gsem).wait()
    pltpu.async_copy(gbuf[0].at[pl.ds(0, _TAIL)], acc.at[tldst],
                     ssem, add=True).wait()

    plsc.subcore_barrier()
    w0 = s * _WPT
    pltpu.sync_copy(acc.at[pl.ds(w0, _WPT)],
                    out_h.at[pl.ds(coff + w0, _WPT)])

    @pl.when(s == 0)
    def _():
        pltpu.sync_copy(acc.at[pl.ds(16 * _WPT, _HALFN - 16 * _WPT)],
                        out_h.at[pl.ds(coff + 16 * _WPT, _HALFN - 16 * _WPT)])


_sc_mesh = plsc.VectorSubcoreMesh(core_axis_name="c", subcore_axis_name="s",
                                  num_cores=_NC, num_subcores=_NS)

_spmm_call = functools.partial(
    pl.kernel,
    out_type=jax.ShapeDtypeStruct((_N, _FP), jnp.float32),
    mesh=_sc_mesh,
    compiler_params=pltpu.CompilerParams(use_tc_tiling_on_sc=False),
    scratch_types=(
        [pltpu.VMEM((2, _CH), jnp.int32)] * _ND
        + [pltpu.VMEM((_CH,), jnp.int32)] * _ND
        + [pltpu.VMEM((_CH, _FP), jnp.float32)] * _ND
        + [pltpu.VMEM((2, _TAIL), jnp.int32),
           pltpu.VMEM((_TAIL,), jnp.int32),
           pltpu.VMEM((_ZB, _FP), jnp.float32),
           pltpu.VMEM_SHARED((_ACC_ROWS, _FP), jnp.float32),
           pltpu.SemaphoreType.DMA,
           pltpu.SemaphoreType.DMA,
           pltpu.SemaphoreType.DMA]
    ),
)(_spmm_body)


def _spmm(eidx, m):
    return _spmm_call(eidx, m)


def _vspec(w=_FP):
    return pl.BlockSpec((_BN, w), lambda i: (i, 0))


def _wspec(shape):
    return pl.BlockSpec(shape, lambda i: (0, 0))


def _k1_body(x_ref, deg_ref, w1, b1, w2, b2, w3, b3,
             h1_ref, h2_ref, h3_ref, dis_ref, dinv_ref):
    xb = x_ref[...]

    def proj(w, b):
        t = jnp.dot(xb, w[...], preferred_element_type=jnp.float32) + b[...]
        return jnp.where(t >= 0, t, 0.01 * t)

    h1_ref[...] = proj(w1, b1)
    h2_ref[...] = proj(w2, b2)
    h3_ref[...] = proj(w3, b3)
    deg0 = deg_ref[...][:, 0:1]
    dis = lax.rsqrt(deg0 + 1.0)
    dinv = jnp.where(deg0 > 0, lax.rsqrt(jnp.maximum(deg0, 1.0)), 0.0)
    dis_ref[...] = jnp.broadcast_to(dis, (_BN, 8))
    dinv_ref[...] = jnp.broadcast_to(dinv, (_BN, 8))


_k1 = pl.pallas_call(
    _k1_body,
    grid=(_NBLK,),
    in_specs=[_vspec(8), _vspec(_FP), _wspec((8, _FP)), _wspec((1, _FP)),
              _wspec((8, _FP)), _wspec((1, _FP)), _wspec((8, _FP)),
              _wspec((1, _FP))],
    out_specs=[_vspec(), _vspec(), _vspec(), _vspec(8), _vspec(8)],
    out_shape=[jax.ShapeDtypeStruct((_N, _FP), jnp.float32)] * 3
    + [jax.ShapeDtypeStruct((_N, 8), jnp.float32)] * 2,
)


def _gcn_a_body(h_ref, w_ref, dis_ref, m_ref, v_ref):
    m = jnp.dot(h_ref[...], w_ref[...], preferred_element_type=jnp.float32)
    m_ref[...] = m
    v_ref[...] = dis_ref[...][:, 0:1] * m


_gcn_a = pl.pallas_call(
    _gcn_a_body,
    grid=(_NBLK,),
    in_specs=[_vspec(), _wspec((_FP, _FP)), _vspec(8)],
    out_specs=[_vspec(), _vspec()],
    out_shape=[jax.ShapeDtypeStruct((_N, _FP), jnp.float32)] * 2,
)


def _gcn_b_body(s_ref, m_ref, dis_ref, b_ref, h_ref):
    d0 = dis_ref[...][:, 0:1]
    h_ref[...] = d0 * s_ref[...] + (d0 * d0) * m_ref[...] + b_ref[...]


_gcn_b = pl.pallas_call(
    _gcn_b_body,
    grid=(_NBLK,),
    in_specs=[_vspec(), _vspec(), _vspec(8), _wspec((1, _FP))],
    out_specs=_vspec(),
    out_shape=jax.ShapeDtypeStruct((_N, _FP), jnp.float32),
)


def _gin_body(s_ref, h_ref, w_ref, b_ref, o_ref):
    t = jnp.dot(h_ref[...] + s_ref[...], w_ref[...],
                preferred_element_type=jnp.float32) + b_ref[...]
    o_ref[...] = t - jnp.tanh(t)


_gin_c = pl.pallas_call(
    _gin_body,
    grid=(_NBLK,),
    in_specs=[_vspec(), _vspec(), _wspec((_FP, _FP)), _wspec((1, _FP))],
    out_specs=_vspec(),
    out_shape=jax.ShapeDtypeStruct((_N, _FP), jnp.float32),
)


def _cheb0_body(t0_ref, w_ref, dinv_ref, o3_ref, v_ref):
    t0 = t0_ref[...]
    o3_ref[...] = jnp.dot(t0, w_ref[...], preferred_element_type=jnp.float32)
    v_ref[...] = dinv_ref[...][:, 0:1] * t0


_cheb0 = pl.pallas_call(
    _cheb0_body,
    grid=(_NBLK,),
    in_specs=[_vspec(), _wspec((_FP, _FP)), _vspec(8)],
    out_specs=[_vspec(), _vspec()],
    out_shape=[jax.ShapeDtypeStruct((_N, _FP), jnp.float32)] * 2,
)


def _cheb1_body(s_ref, o3in_ref, w_ref, dinv_ref, o3_ref, t1_ref, v_ref):
    d0 = dinv_ref[...][:, 0:1]
    t1 = -d0 * s_ref[...]
    o3_ref[...] = o3in_ref[...] + jnp.dot(t1, w_ref[...],
                                          preferred_element_type=jnp.float32)
    t1_ref[...] = t1
    v_ref[...] = d0 * t1


_cheb1 = pl.pallas_call(
    _cheb1_body,
    grid=(_NBLK,),
    in_specs=[_vspec(), _vspec(), _wspec((_FP, _FP)), _vspec(8)],
    out_specs=[_vspec(), _vspec(), _vspec()],
    out_shape=[jax.ShapeDtypeStruct((_N, _FP), jnp.float32)] * 3,
)


def _chebk_body(s_ref, tp_ref, o3in_ref, w_ref, dinv_ref,
                o3_ref, tk_ref, v_ref):
    d0 = dinv_ref[...][:, 0:1]
    tk = -2.0 * d0 * s_ref[...] - tp_ref[...]
    o3_ref[...] = o3in_ref[...] + jnp.dot(tk, w_ref[...],
                                          preferred_element_type=jnp.float32)
    tk_ref[...] = tk
    v_ref[...] = d0 * tk


_chebk = pl.pallas_call(
    _chebk_body,
    grid=(_NBLK,),
    in_specs=[_vspec(), _vspec(), _vspec(), _wspec((_FP, _FP)), _vspec(8)],
    out_specs=[_vspec(), _vspec(), _vspec()],
    out_shape=[jax.ShapeDtypeStruct((_N, _FP), jnp.float32)] * 3,
)


def _cheb9_body(s_ref, tp_ref, o3in_ref, w_ref, dinv_ref, cb_ref, o3_ref):
    d0 = dinv_ref[...][:, 0:1]
    t9 = -2.0 * d0 * s_ref[...] - tp_ref[...]
    o3_ref[...] = (o3in_ref[...]
                   + jnp.dot(t9, w_ref[...], preferred_element_type=jnp.float32)
                   + cb_ref[...])


_cheb9 = pl.pallas_call(
    _cheb9_body,
    grid=(_NBLK,),
    in_specs=[_vspec(), _vspec(), _vspec(), _wspec((_FP, _FP)), _vspec(8),
              _wspec((1, _FP))],
    out_specs=_vspec(),
    out_shape=jax.ShapeDtypeStruct((_N, _FP), jnp.float32),
)


def _pool_body(h_ref, b_ref, add_ref, cnt_ref, mx_ref):
    gb = pl.program_id(0)
    nb = pl.program_id(1)

    @pl.when(nb == 0)
    def _():
        add_ref[...] = jnp.zeros_like(add_ref)
        cnt_ref[...] = jnp.zeros_like(cnt_ref)
        mx_ref[...] = jnp.full_like(mx_ref, -1e30)

    h = h_ref[...]
    bb = b_ref[...]
    gids = gb * 8 + lax.broadcasted_iota(jnp.int32, (1, 8), 1)
    p = bb == gids
    pf = p.astype(jnp.float32)
    add_ref[...] += lax.dot_general(pf, h, (((0,), (0,)), ((), ())),
                                    preferred_element_type=jnp.float32)
    cnt_ref[...] += lax.dot_general(pf, jnp.ones((_BN, 1), jnp.float32),
                                    (((0,), (0,)), ((), ())),
                                    preferred_element_type=jnp.float32)
    for j in range(8):
        mj = p[:, j:j + 1]
        cand = jnp.max(jnp.where(mj, h, -1e30), axis=0, keepdims=True)
        mx_ref[j:j + 1, :] = jnp.maximum(mx_ref[j:j + 1, :], cand)


_pool = pl.pallas_call(
    _pool_body,
    grid=(_G // 8, _NBLK),
    in_specs=[pl.BlockSpec((_BN, _FP), lambda g, i: (i, 0)),
              pl.BlockSpec((_BN, 1), lambda g, i: (i, 0))],
    out_specs=[pl.BlockSpec((8, _FP), lambda g, i: (g, 0)),
               pl.BlockSpec((8, 1), lambda g, i: (g, 0)),
               pl.BlockSpec((8, _FP), lambda g, i: (g, 0))],
    out_shape=[jax.ShapeDtypeStruct((_G, _FP), jnp.float32),
               jax.ShapeDtypeStruct((_G, 1), jnp.float32),
               jax.ShapeDtypeStruct((_G, _FP), jnp.float32)],
)


def _fin_body(a1, c1, m1, a2, c2, m2, a3, c3, m3, fw, fb, out_ref):
    def part(a_ref, c_ref, m_ref):
        a = a_ref[...]
        c = c_ref[...]
        mean = a / jnp.maximum(c, 1.0)
        mx = jnp.where(c > 0, m_ref[...], 0.0)
        return jnp.concatenate([a, mean, mx], axis=1)

    feats = jnp.concatenate([part(a1, c1, m1), part(a2, c2, m2),
                             part(a3, c3, m3)], axis=1)
    logits = jnp.dot(feats, fw[...], preferred_element_type=jnp.float32) \
        + fb[...]
    mxl = jnp.max(logits, axis=-1, keepdims=True)
    sh = logits - mxl
    out_ref[...] = sh - jnp.log(jnp.sum(jnp.exp(sh), axis=-1, keepdims=True))


_fin = pl.pallas_call(
    _fin_body,
    out_shape=jax.ShapeDtypeStruct((_G, 1), jnp.float32),
)


def _pad_w(w):
    fi, fo = w.shape
    return jnp.pad(w, ((0, 0), (0, _FP - fo))) if fi == _FP else \
        jnp.pad(w, ((0, 8 - fi), (0, _FP - fo)))


def _pad_b(b):
    return jnp.pad(b, (0, _FP - b.shape[0])).reshape(1, _FP)


def kernel(x, edge_index, batch, lin1_W, lin1_b, gcn_W, gcn_b, lin2_W, lin2_b,
           gin_W, gin_b, lin3_W, lin3_b, cheb_W, cheb_b, fin_W, fin_b):
    eidx = edge_index.astype(jnp.int32)
    batch2 = batch.astype(jnp.int32).reshape(_N, 1)
    x_pad = jnp.pad(x, ((0, 0), (0, 8 - x.shape[1])))

    gw = [jnp.pad(gcn_W[i], ((0, 4), (0, 4))) for i in range(_NLAYERS)]
    gb = [_pad_b(gcn_b[i]) for i in range(_NLAYERS)]
    iw = [jnp.pad(gin_W[i], ((0, 4), (0, 4))) for i in range(_NLAYERS)]
    ib = [_pad_b(gin_b[i]) for i in range(_NLAYERS)]
    cw = [jnp.pad(cheb_W[k], ((0, 4), (0, 4))) for k in range(_KCHEB)]
    cbp = _pad_b(cheb_b)
    fwp = jnp.pad(fin_W.reshape(9, 60, 1), ((0, 0), (0, 4), (0, 0))) \
        .reshape(9 * _FP, 1)
    fbp = fin_b.reshape(1, 1)

    ones64 = jnp.ones((_N, _FP), jnp.float32)
    deg64 = _spmm(eidx, ones64)
    h1, h2, h3, dis8, dinv8 = _k1(x_pad, deg64, _pad_w(lin1_W), _pad_b(lin1_b),
                                  _pad_w(lin2_W), _pad_b(lin2_b),
                                  _pad_w(lin3_W), _pad_b(lin3_b))

    h = h1
    for i in range(_NLAYERS):
        m, v = _gcn_a(h, gw[i], dis8)
        agg = _spmm(eidx, v)
        h = _gcn_b(agg, m, dis8, gb[i])
    p1 = _pool(h, batch2)

    h = h2
    for i in range(_NLAYERS):
        agg = _spmm(eidx, h)
        h = _gin_c(agg, h, iw[i], ib[i])
    p2 = _pool(h, batch2)

    o3, v = _cheb0(h3, cw[0], dinv8)
    agg = _spmm(eidx, v)
    o3, t_im1, v = _cheb1(agg, o3, cw[1], dinv8)
    t_im2 = h3
    for k in range(2, _KCHEB):
        agg = _spmm(eidx, v)
        if k < _KCHEB - 1:
            o3, tk, v = _chebk(agg, t_im2, o3, cw[k], dinv8)
            t_im2, t_im1 = t_im1, tk
        else:
            o3 = _cheb9(agg, t_im2, o3, cw[k], dinv8, cbp)
    p3 = _pool(o3, batch2)

    return _fin(p1[0], p1[1], p1[2], p2[0], p2[1], p2[2],
                p3[0], p3[1], p3[2], fwp, fbp)
